# baseline (speedup 1.0000x reference)
import functools

import jax
import jax.numpy as jnp
from jax import lax
from jax.experimental import pallas as pl
from jax.experimental.pallas import tpu as pltpu
from jax.experimental.pallas import tpu_sc as plsc

NUM_POSITIONS = 1024
HIDDEN = 768
BATCH = 64
SEQ = 1024

NC = 2
NS = 16
NW = NC * NS

TOTAL = BATCH * SEQ
BPW = TOTAL // NW
CHUNK = 32
NCHUNK = BPW // CHUNK
NB = 4

_mesh = plsc.VectorSubcoreMesh(core_axis_name="c", subcore_axis_name="s")


@functools.partial(
    pl.kernel,
    mesh=_mesh,
    out_type=jax.ShapeDtypeStruct((TOTAL, HIDDEN), jnp.float32),
    scratch_types=[
        pltpu.VMEM((NCHUNK, CHUNK), jnp.int32),
        pltpu.VMEM((NB, CHUNK, HIDDEN), jnp.float32),
    ] + [pltpu.SemaphoreType.DMA] * (2 * NB),
)
def _gather_rows(ids_hbm, table_hbm, out_hbm, idx_v, rows_v, *sems):
    gs = sems[:NB]
    ws = sems[NB:]
    wid = lax.axis_index("s") * NC + lax.axis_index("c")
    base = wid * BPW
    pltpu.sync_copy(ids_hbm.at[wid], idx_v)

    def start_gather(ci, b):
        pltpu.async_copy(table_hbm.at[idx_v.at[ci]], rows_v.at[b], gs[b])

    def wait_gather(b):
        pltpu.make_async_copy(
            table_hbm.at[idx_v.at[0]], rows_v.at[b], gs[b]).wait()

    def start_write(ci, b):
        pltpu.async_copy(
            rows_v.at[b], out_hbm.at[pl.ds(base + ci * CHUNK, CHUNK)], ws[b])

    def wait_write(b):
        pltpu.make_async_copy(
            rows_v.at[b], out_hbm.at[pl.ds(base, CHUNK)], ws[b]).wait()

    def step(o, carry):
        later = o > 0
        for r in range(NB):
            b = r

            @pl.when(later)
            def _(b=b):
                wait_write(b)

            start_gather(NB * o + r, b)
            bj = (r + 1) % NB
            if r == NB - 1:
                wait_gather(bj)
                start_write(NB * o, bj)
            else:
                @pl.when(later)
                def _(r=r, bj=bj):
                    wait_gather(bj)
                    start_write(NB * (o - 1) + r + 1, bj)
        return carry

    lax.fori_loop(0, NCHUNK // NB, step, 0)
    for j in range(NCHUNK - NB + 1, NCHUNK):
        wait_gather(j % NB)
        start_write(j, j % NB)
    for b in range(NB):
        wait_write(b)


def kernel(position_ids, table):
    ids = jnp.reshape(position_ids.astype(jnp.int32), (NW, NCHUNK, CHUNK))
    out = _gather_rows(ids, table)
    return jnp.reshape(out, (BATCH, SEQ, HIDDEN))
